# baseline (speedup 1.0000x reference)
import functools

import jax
import jax.numpy as jnp
from jax import lax
from jax.experimental import pallas as pl
from jax.experimental.pallas import tpu as pltpu
from jax.experimental.pallas import tpu_sc as plsc

VOCAB = 1000000
EMB = 64
B = 16384

_info = plsc.get_sparse_core_info()
NC, NS, L = _info.num_cores, _info.num_subcores, _info.num_lanes
NW = NC * NS
BPW = B // NW
NGRP = BPW // L


@functools.partial(
    pl.kernel,
    mesh=plsc.VectorSubcoreMesh(core_axis_name="c", subcore_axis_name="s"),
    out_type=jax.ShapeDtypeStruct((B, EMB), jnp.float32),
    scratch_types=[
        pltpu.VMEM((BPW,), jnp.int32),
        pltpu.VMEM((2, L, EMB, L), jnp.float32),
        pltpu.VMEM((BPW, EMB), jnp.float32),
        pltpu.SemaphoreType.DMA,
    ],
    compiler_params=pltpu.CompilerParams(
        use_tc_tiling_on_sc=False, needs_layout_passes=False
    ),
)
def _gather_kernel(tableT_hbm, idx_hbm, out_hbm, idx_v, win_v, rows_v, sem):
    wid = lax.axis_index("s") * NC + lax.axis_index("c")
    base = wid * BPW
    pltpu.sync_copy(idx_hbm.at[pl.ds(base, BPW)], idx_v)

    iota = lax.iota(jnp.int32, L)

    def fire_group(g):
        slot = lax.rem(g, 2)
        vec = idx_v[pl.ds(g * L, L)]
        vbase = (vec >> 4) << 4
        for l in range(L):
            off = pl.multiple_of(vbase[l], L)
            pltpu.make_async_copy(
                tableT_hbm.at[:, pl.ds(off, L)],
                win_v.at[slot, l],
                sem,
            ).start()

    def drain_group(g):
        slot = lax.rem(g, 2)
        for l in range(L):
            pltpu.make_async_copy(
                tableT_hbm.at[:, pl.ds(0, L)],
                win_v.at[slot, l],
                sem,
            ).wait()

    def extract_group(g):
        slot = lax.rem(g, 2)
        vec = idx_v[pl.ds(g * L, L)]
        ri_vec = vec & 15
        for l in range(L):
            ri = jnp.full((L,), ri_vec[l], jnp.int32)
            for k in range(EMB // L):
                e_vec = iota + k * L
                val = plsc.load_gather(win_v.at[slot, l], [e_vec, ri])
                rows_v[g * L + l, pl.ds(k * L, L)] = val

    fire_group(0)

    def body(g, _):
        fire_group(g + 1)
        drain_group(g)
        extract_group(g)
        return 0

    lax.fori_loop(0, NGRP - 1, body, 0, unroll=False)
    drain_group(NGRP - 1)
    extract_group(NGRP - 1)

    pltpu.sync_copy(rows_v, out_hbm.at[pl.ds(base, BPW)])


def kernel(input, hidden, cell, table):
    idx = input.astype(jnp.int32)
    out = _gather_kernel(table.T, idx)
    return out[:, None, :]


# speedup vs baseline: 20.1202x; 20.1202x over previous
import functools

import jax
import jax.numpy as jnp
from jax import lax
from jax.experimental import pallas as pl
from jax.experimental.pallas import tpu as pltpu
from jax.experimental.pallas import tpu_sc as plsc

VOCAB = 1000000
EMB = 64
B = 16384
LANE = 128

_info = plsc.get_sparse_core_info()
NC, NS, L = _info.num_cores, _info.num_subcores, _info.num_lanes
NW = NC * NS
BPW = B // NW
SG = 2
NSG = L // SG


@functools.partial(
    pl.kernel,
    mesh=plsc.VectorSubcoreMesh(core_axis_name="c", subcore_axis_name="s"),
    out_type=jax.ShapeDtypeStruct((B, EMB), jnp.float32),
    scratch_types=[
        pltpu.VMEM((BPW,), jnp.int32),
        pltpu.VMEM((2, SG, EMB, LANE), jnp.float32),
        pltpu.VMEM((BPW, EMB), jnp.float32),
        pltpu.SemaphoreType.DMA,
    ],
    compiler_params=pltpu.CompilerParams(needs_layout_passes=False),
)
def _gather_kernel(tableT_hbm, idx_hbm, out_hbm, idx_v, win_v, rows_v, sem):
    wid = lax.axis_index("s") * NC + lax.axis_index("c")
    base = wid * BPW
    pltpu.sync_copy(idx_hbm.at[pl.ds(base, BPW)], idx_v)

    iota = lax.iota(jnp.int32, L)

    def body(g, _):
        vec = idx_v[pl.ds(g * L, L)]
        vbase = (vec >> 7) << 7
        ri_vec = vec & (LANE - 1)

        def fire(j):
            for l in range(SG):
                off = pl.multiple_of(vbase[j * SG + l], LANE)
                pltpu.make_async_copy(
                    tableT_hbm.at[:, pl.ds(off, LANE)],
                    win_v.at[j % 2, l],
                    sem,
                ).start()

        def drain():
            for l in range(SG):
                pltpu.make_async_copy(
                    tableT_hbm.at[:, pl.ds(0, LANE)],
                    win_v.at[0, l],
                    sem,
                ).wait()

        def extract(j):
            for l in range(SG):
                ri = jnp.full((L,), ri_vec[j * SG + l], jnp.int32)
                for k in range(EMB // L):
                    e_vec = iota + k * L
                    val = plsc.load_gather(win_v.at[j % 2, l], [e_vec, ri])
                    rows_v[g * L + j * SG + l, pl.ds(k * L, L)] = val

        fire(0)
        for j in range(NSG):
            if j + 1 < NSG:
                fire(j + 1)
            drain()
            extract(j)
        return 0

    lax.fori_loop(0, BPW // L, body, 0, unroll=False)

    pltpu.sync_copy(rows_v, out_hbm.at[pl.ds(base, BPW)])


def kernel(input, hidden, cell, table):
    idx = input.astype(jnp.int32)
    out = _gather_kernel(table.T, idx)
    return out[:, None, :]


# speedup vs baseline: 22.5661x; 1.1216x over previous
import functools

import jax
import jax.numpy as jnp
from jax import lax
from jax.experimental import pallas as pl
from jax.experimental.pallas import tpu as pltpu
from jax.experimental.pallas import tpu_sc as plsc

VOCAB = 1000000
EMB = 64
B = 16384
LANE = 128

_info = plsc.get_sparse_core_info()
NC, NS, L = _info.num_cores, _info.num_subcores, _info.num_lanes
NW = NC * NS
BPW = B // NW
SG = 2
NSG = L // SG


@functools.partial(
    pl.kernel,
    mesh=plsc.VectorSubcoreMesh(core_axis_name="c", subcore_axis_name="s"),
    out_type=jax.ShapeDtypeStruct((B, EMB), jnp.float32),
    scratch_types=[
        pltpu.VMEM((BPW,), jnp.int32),
        pltpu.VMEM((3, SG, EMB, LANE), jnp.float32),
        pltpu.VMEM((BPW, EMB), jnp.float32),
        pltpu.SemaphoreType.DMA,
    ],
    compiler_params=pltpu.CompilerParams(needs_layout_passes=False),
)
def _gather_kernel(tableT_hbm, idx_hbm, out_hbm, idx_v, win_v, rows_v, sem):
    wid = lax.axis_index("s") * NC + lax.axis_index("c")
    base = wid * BPW
    pltpu.sync_copy(idx_hbm.at[pl.ds(base, BPW)], idx_v)

    iota = lax.iota(jnp.int32, L)

    def body(g, _):
        vec = idx_v[pl.ds(g * L, L)]
        vbase = (vec >> 7) << 7
        ri_vec = vec & (LANE - 1)

        def fire(j):
            for l in range(SG):
                off = pl.multiple_of(vbase[j * SG + l], LANE)
                pltpu.make_async_copy(
                    tableT_hbm.at[:, pl.ds(off, LANE)],
                    win_v.at[j % 3, l],
                    sem,
                ).start()

        def drain():
            for l in range(SG):
                pltpu.make_async_copy(
                    tableT_hbm.at[:, pl.ds(0, LANE)],
                    win_v.at[0, l],
                    sem,
                ).wait()

        def extract(j):
            for l in range(SG):
                ri = jnp.full((L,), ri_vec[j * SG + l], jnp.int32)
                for k in range(EMB // L):
                    e_vec = iota + k * L
                    val = plsc.load_gather(win_v.at[j % 3, l], [e_vec, ri])
                    rows_v[g * L + j * SG + l, pl.ds(k * L, L)] = val

        fire(0)
        fire(1)
        for j in range(NSG):
            if j + 2 < NSG:
                fire(j + 2)
            drain()
            extract(j)
        return 0

    lax.fori_loop(0, BPW // L, body, 0, unroll=False)

    pltpu.sync_copy(rows_v, out_hbm.at[pl.ds(base, BPW)])


def kernel(input, hidden, cell, table):
    idx = input.astype(jnp.int32)
    out = _gather_kernel(table.T, idx)
    return out[:, None, :]
